# Initial kernel scaffold; baseline (speedup 1.0000x reference)
import functools

import jax
import jax.numpy as jnp
from jax.experimental import pallas as pl
from jax.experimental.pallas import tpu as pltpu


def _moe_dense_kernel(h_ref, gw_ref, gp_ref, up_ref, dp_ref, out_ref,
                      hbf_scr, wcol_scr):
    e = pl.program_id(0)
    f = pl.program_id(1)

    @pl.when((e == 0) & (f == 0))
    def _init():
        out_ref[...] = jnp.zeros_like(out_ref)
        hbf_scr[...] = h_ref[...].astype(jnp.bfloat16)

    @pl.when(f == 0)
    def _router():
        h = h_ref[...]
        logits = jax.lax.dot_general(
            h, gw_ref[...], (((1,), (1,)), ((), ())),
            preferred_element_type=jnp.float32)
        num_e = logits.shape[1]
        logits = logits - jnp.max(logits, axis=1, keepdims=True)
        ex = jnp.exp(logits)
        probs = ex / jnp.sum(ex, axis=1, keepdims=True)
        iota = jax.lax.broadcasted_iota(jnp.int32, probs.shape, 1)
        m1 = jnp.max(probs, axis=1, keepdims=True)
        i1 = jnp.min(jnp.where(probs == m1, iota, num_e), axis=1,
                     keepdims=True)
        probs2 = jnp.where(iota == i1, -jnp.inf, probs)
        m2 = jnp.max(probs2, axis=1, keepdims=True)
        i2 = jnp.min(jnp.where(probs2 == m2, iota, num_e), axis=1,
                     keepdims=True)
        denom = jnp.maximum(m1 + m2, 1e-9)
        wcol = (m1 * (i1 == e) + m2 * (i2 == e)) / denom
        wcol_scr[...] = wcol

    hbf = hbf_scr[...]
    wg = gp_ref[0].astype(jnp.bfloat16)
    wu = up_ref[0].astype(jnp.bfloat16)
    wd = dp_ref[0].astype(jnp.bfloat16)
    g = jax.lax.dot_general(hbf, wg, (((1,), (1,)), ((), ())),
                            preferred_element_type=jnp.float32)
    u = jax.lax.dot_general(hbf, wu, (((1,), (1,)), ((), ())),
                            preferred_element_type=jnp.float32)
    act = g * jax.nn.sigmoid(g) * u * wcol_scr[...]
    act_bf = act.astype(jnp.bfloat16)
    y = jax.lax.dot_general(act_bf, wd, (((1,), (1,)), ((), ())),
                            preferred_element_type=jnp.float32)
    out_ref[...] += y


def kernel(hidden_states, gate_w, gate_proj_w, up_proj_w, down_proj_w):
    orig_shape = hidden_states.shape
    H = orig_shape[-1]
    h = hidden_states.reshape(-1, H)
    BT = h.shape[0]
    E, F, _ = gate_proj_w.shape
    FB = 256
    n_f = F // FB

    out = pl.pallas_call(
        _moe_dense_kernel,
        grid=(E, n_f),
        in_specs=[
            pl.BlockSpec((BT, H), lambda e, f: (0, 0)),
            pl.BlockSpec((E, H), lambda e, f: (0, 0)),
            pl.BlockSpec((1, FB, H), lambda e, f: (e, f, 0)),
            pl.BlockSpec((1, FB, H), lambda e, f: (e, f, 0)),
            pl.BlockSpec((1, H, FB), lambda e, f: (e, 0, f)),
        ],
        out_specs=pl.BlockSpec((BT, H), lambda e, f: (0, 0)),
        out_shape=jax.ShapeDtypeStruct((BT, H), jnp.float32),
        scratch_shapes=[
            pltpu.VMEM((BT, H), jnp.bfloat16),
            pltpu.VMEM((BT, 1), jnp.float32),
        ],
        compiler_params=pltpu.CompilerParams(
            dimension_semantics=("arbitrary", "arbitrary"),
        ),
    )(h, gate_w, gate_proj_w, up_proj_w, down_proj_w)
    return out.reshape(orig_shape)


# speedup vs baseline: 1.6425x; 1.6425x over previous
import jax
import jax.numpy as jnp
from jax.experimental import pallas as pl
from jax.experimental.pallas import tpu as pltpu


def _moe_dense_kernel(h_ref, gw_ref, gp_ref, up_ref, dp_ref, out_ref,
                      hbf_scr, wcol_scr):
    e = pl.program_id(1)
    f = pl.program_id(2)

    @pl.when((e == 0) & (f == 0))
    def _init():
        out_ref[...] = jnp.zeros_like(out_ref)
        hbf_scr[...] = h_ref[...].astype(jnp.bfloat16)

    @pl.when(f == 0)
    def _router():
        h = h_ref[...]
        logits = jax.lax.dot_general(
            h, gw_ref[...], (((1,), (1,)), ((), ())),
            preferred_element_type=jnp.float32)
        num_e = logits.shape[1]
        logits = logits - jnp.max(logits, axis=1, keepdims=True)
        ex = jnp.exp(logits)
        probs = ex / jnp.sum(ex, axis=1, keepdims=True)
        iota = jax.lax.broadcasted_iota(jnp.int32, probs.shape, 1)
        m1 = jnp.max(probs, axis=1, keepdims=True)
        i1 = jnp.min(jnp.where(probs == m1, iota, num_e), axis=1,
                     keepdims=True)
        probs2 = jnp.where(iota == i1, -jnp.inf, probs)
        m2 = jnp.max(probs2, axis=1, keepdims=True)
        i2 = jnp.min(jnp.where(probs2 == m2, iota, num_e), axis=1,
                     keepdims=True)
        denom = jnp.maximum(m1 + m2, 1e-9)
        wcol = (m1 * (i1 == e) + m2 * (i2 == e)) / denom
        wcol_scr[...] = wcol

    hbf = hbf_scr[...]
    wg = gp_ref[0].astype(jnp.bfloat16)
    wu = up_ref[0].astype(jnp.bfloat16)
    wd = dp_ref[0].astype(jnp.bfloat16)
    g = jax.lax.dot_general(hbf, wg, (((1,), (1,)), ((), ())),
                            preferred_element_type=jnp.float32)
    u = jax.lax.dot_general(hbf, wu, (((1,), (1,)), ((), ())),
                            preferred_element_type=jnp.float32)
    act = g * jax.nn.sigmoid(g) * u * wcol_scr[...]
    act_bf = act.astype(jnp.bfloat16)
    y = jax.lax.dot_general(act_bf, wd, (((1,), (1,)), ((), ())),
                            preferred_element_type=jnp.float32)
    out_ref[...] += y


def kernel(hidden_states, gate_w, gate_proj_w, up_proj_w, down_proj_w):
    orig_shape = hidden_states.shape
    H = orig_shape[-1]
    h = hidden_states.reshape(-1, H)
    BT = h.shape[0]
    E, F, _ = gate_proj_w.shape
    FB = 256
    n_f = F // FB
    MB = 2048
    n_m = BT // MB

    out = pl.pallas_call(
        _moe_dense_kernel,
        grid=(n_m, E, n_f),
        in_specs=[
            pl.BlockSpec((MB, H), lambda m, e, f: (m, 0)),
            pl.BlockSpec((E, H), lambda m, e, f: (0, 0)),
            pl.BlockSpec((1, FB, H), lambda m, e, f: (e, f, 0)),
            pl.BlockSpec((1, FB, H), lambda m, e, f: (e, f, 0)),
            pl.BlockSpec((1, H, FB), lambda m, e, f: (e, 0, f)),
        ],
        out_specs=pl.BlockSpec((MB, H), lambda m, e, f: (m, 0)),
        out_shape=jax.ShapeDtypeStruct((BT, H), jnp.float32),
        scratch_shapes=[
            pltpu.VMEM((MB, H), jnp.bfloat16),
            pltpu.VMEM((MB, 1), jnp.float32),
        ],
        compiler_params=pltpu.CompilerParams(
            dimension_semantics=("arbitrary", "arbitrary", "arbitrary"),
        ),
    )(h, gate_w, gate_proj_w, up_proj_w, down_proj_w)
    return out.reshape(orig_shape)


# speedup vs baseline: 2.4378x; 1.4842x over previous
import functools

import jax
import jax.numpy as jnp
from jax import lax
from jax.experimental import pallas as pl
from jax.experimental.pallas import tpu as pltpu
from jax.experimental.pallas import tpu_sc as plsc

_BM = 1024
_FB = 256
_CHUNK = 32
_GPAD = 128


def _router_kernel(h_ref, gw_ref, p0_ref, p1_ref, g1_ref, g2_ref,
                   eid_ref, nblk_ref, *, bm, nblk_max):
    h = h_ref[...]
    bt = h.shape[0]
    num_e = gw_ref.shape[0]
    logits = jax.lax.dot_general(h, gw_ref[...], (((1,), (1,)), ((), ())),
                                 preferred_element_type=jnp.float32)
    logits = logits - jnp.max(logits, axis=1, keepdims=True)
    ex = jnp.exp(logits)
    probs = ex / jnp.sum(ex, axis=1, keepdims=True)
    iota = jax.lax.broadcasted_iota(jnp.int32, probs.shape, 1)
    m1 = jnp.max(probs, axis=1, keepdims=True)
    i1 = jnp.min(jnp.where(probs == m1, iota, num_e), axis=1, keepdims=True)
    probs2 = jnp.where(iota == i1, -jnp.inf, probs)
    m2 = jnp.max(probs2, axis=1, keepdims=True)
    i2 = jnp.min(jnp.where(probs2 == m2, iota, num_e), axis=1, keepdims=True)
    denom = jnp.maximum(m1 + m2, 1e-9)
    g1_ref[...] = m1 / denom
    g2_ref[...] = m2 / denom

    oh0 = (iota == i1).astype(jnp.float32)
    oh1 = (iota == i2).astype(jnp.float32)
    oh = jnp.concatenate([oh0, oh1], axis=1).astype(jnp.bfloat16)
    ch = min(512, bt)
    r = jnp.zeros((bt, 2 * num_e), jnp.float32)
    for c in range(bt // ch):
        col = jax.lax.broadcasted_iota(jnp.int32, (bt, ch), 1) + c * ch
        row = jax.lax.broadcasted_iota(jnp.int32, (bt, ch), 0)
        ltri = (col < row).astype(jnp.bfloat16)
        r = r + jax.lax.dot_general(
            ltri, oh[c * ch:(c + 1) * ch, :], (((1,), (0,)), ((), ())),
            preferred_element_type=jnp.float32)
    r0 = r[:, :num_e]
    r1 = r[:, num_e:]
    cnt0 = jnp.sum(oh0, axis=0, keepdims=True)
    cnt1 = jnp.sum(oh1, axis=0, keepdims=True)
    count = cnt0 + cnt1
    nblk_e = jnp.floor((count + (bm - 1)) / bm)
    tri = (jax.lax.broadcasted_iota(jnp.int32, (num_e, num_e), 0)
           <= jax.lax.broadcasted_iota(jnp.int32, (num_e, num_e), 1)
           ).astype(jnp.float32)
    cum = jax.lax.dot_general(nblk_e, tri, (((1,), (0,)), ((), ())),
                              preferred_element_type=jnp.float32)
    pad_off = (cum - nblk_e) * bm
    p0 = jnp.sum(oh0 * (pad_off + r0), axis=1, keepdims=True)
    p1 = jnp.sum(oh1 * (pad_off + cnt0 + r1), axis=1, keepdims=True)
    p0_ref[...] = p0.astype(jnp.int32)
    p1_ref[...] = p1.astype(jnp.int32)

    m_iota = jax.lax.broadcasted_iota(jnp.int32, (nblk_max, num_e), 0)
    cum_b = jnp.broadcast_to(cum, (nblk_max, num_e))
    eid = jnp.sum((cum_b <= m_iota.astype(jnp.float32)).astype(jnp.int32),
                  axis=1, keepdims=True)
    eid_ref[...] = jnp.minimum(eid, num_e - 1)
    nblk_ref[...] = cum[:, num_e - 1:num_e].astype(jnp.int32)


def _make_scatter(nw, rows_per_w, hx, npad):
    mesh = plsc.VectorSubcoreMesh(core_axis_name="c", subcore_axis_name="s")

    @functools.partial(
        pl.kernel, mesh=mesh,
        out_type=jax.ShapeDtypeStruct((npad, hx), jnp.float32),
        scratch_types=[
            pltpu.VMEM((_CHUNK,), jnp.int32),
            pltpu.VMEM((_CHUNK,), jnp.int32),
            pltpu.VMEM((_CHUNK, hx), jnp.float32),
            pltpu.VMEM((_CHUNK, hx), jnp.float32),
            pltpu.SemaphoreType.DMA,
            pltpu.SemaphoreType.DMA,
        ])
    def _scatter(hx0_hbm, hx1_hbm, p0_hbm, p1_hbm, out_hbm,
                 idx0_v, idx1_v, rows0_v, rows1_v, sem0, sem1):
        wid = lax.axis_index("s") * 2 + lax.axis_index("c")
        base = wid * rows_per_w
        for c in range(rows_per_w // _CHUNK):
            off = base + c * _CHUNK
            pltpu.sync_copy(p0_hbm.at[pl.ds(off, _CHUNK)], idx0_v)
            pltpu.sync_copy(p1_hbm.at[pl.ds(off, _CHUNK)], idx1_v)
            pltpu.sync_copy(hx0_hbm.at[pl.ds(off, _CHUNK)], rows0_v)
            pltpu.sync_copy(hx1_hbm.at[pl.ds(off, _CHUNK)], rows1_v)
            cp0 = pltpu.async_copy(rows0_v, out_hbm.at[idx0_v], sem0)
            cp1 = pltpu.async_copy(rows1_v, out_hbm.at[idx1_v], sem1)
            cp0.wait()
            cp1.wait()

    return _scatter


def _make_combine(nw, rows_per_w, h):
    mesh = plsc.VectorSubcoreMesh(core_axis_name="c", subcore_axis_name="s")
    bt = nw * rows_per_w

    @functools.partial(
        pl.kernel, mesh=mesh,
        out_type=jax.ShapeDtypeStruct((bt, h), jnp.float32),
        scratch_types=[
            pltpu.VMEM((_CHUNK,), jnp.int32),
            pltpu.VMEM((_CHUNK,), jnp.int32),
            pltpu.VMEM((_CHUNK, h), jnp.float32),
            pltpu.VMEM((_CHUNK, h), jnp.float32),
            pltpu.SemaphoreType.DMA,
            pltpu.SemaphoreType.DMA,
        ])
    def _combine(y_hbm, p0_hbm, p1_hbm, out_hbm,
                 idx0_v, idx1_v, r0_v, r1_v, sem0, sem1):
        wid = lax.axis_index("s") * 2 + lax.axis_index("c")
        base = wid * rows_per_w
        for c in range(rows_per_w // _CHUNK):
            off = base + c * _CHUNK
            pltpu.sync_copy(p0_hbm.at[pl.ds(off, _CHUNK)], idx0_v)
            pltpu.sync_copy(p1_hbm.at[pl.ds(off, _CHUNK)], idx1_v)
            cp0 = pltpu.async_copy(y_hbm.at[idx0_v], r0_v, sem0)
            cp1 = pltpu.async_copy(y_hbm.at[idx1_v], r1_v, sem1)
            cp0.wait()
            cp1.wait()

            def row_body(i, _):
                def lane_body(j, _):
                    sl = pl.ds(j * 16, 16)
                    r0_v[i, sl] = r0_v[i, sl] + r1_v[i, sl]
                    return 0
                lax.fori_loop(0, h // 16, lane_body, 0, unroll=8)
                return 0
            lax.fori_loop(0, _CHUNK, row_body, 0)
            pltpu.sync_copy(r0_v, out_hbm.at[pl.ds(off, _CHUNK)])

    return _combine


def _gmm_kernel(eid_ref, nblk_ref, a_ref, wg_ref, wu_ref, wd_ref, y_ref,
                abf_scr, gate_scr, *, h):
    m = pl.program_id(0)
    f = pl.program_id(1)
    active = m < nblk_ref[0]

    @pl.when(active & (f == 0))
    def _load_a():
        abf_scr[...] = a_ref[:, :h].astype(jnp.bfloat16)
        gate_scr[...] = a_ref[:, h:h + 1]

    @pl.when(active)
    def _compute():
        abf = abf_scr[...]
        wg = wg_ref[0].astype(jnp.bfloat16)
        wu = wu_ref[0].astype(jnp.bfloat16)
        wd = wd_ref[0].astype(jnp.bfloat16)
        g = jax.lax.dot_general(abf, wg, (((1,), (1,)), ((), ())),
                                preferred_element_type=jnp.float32)
        u = jax.lax.dot_general(abf, wu, (((1,), (1,)), ((), ())),
                                preferred_element_type=jnp.float32)
        act = g * jax.nn.sigmoid(g) * u * gate_scr[...]
        y = jax.lax.dot_general(act.astype(jnp.bfloat16), wd,
                                (((1,), (1,)), ((), ())),
                                preferred_element_type=jnp.float32)

        @pl.when(f == 0)
        def _set():
            y_ref[...] = y

        @pl.when(f != 0)
        def _acc():
            y_ref[...] += y


def kernel(hidden_states, gate_w, gate_proj_w, up_proj_w, down_proj_w):
    orig_shape = hidden_states.shape
    H = orig_shape[-1]
    h = hidden_states.reshape(-1, H)
    BT = h.shape[0]
    E, F, _ = gate_proj_w.shape
    K = 2
    nblk_max = (BT * K) // _BM + E
    npad = nblk_max * _BM
    n_f = F // _FB
    HX = H + _GPAD
    NW = 32
    rows_per_w = BT // NW

    p0, p1, g1, g2, eid, nblk = pl.pallas_call(
        functools.partial(_router_kernel, bm=_BM, nblk_max=nblk_max),
        out_shape=[
            jax.ShapeDtypeStruct((BT, 1), jnp.int32),
            jax.ShapeDtypeStruct((BT, 1), jnp.int32),
            jax.ShapeDtypeStruct((BT, 1), jnp.float32),
            jax.ShapeDtypeStruct((BT, 1), jnp.float32),
            jax.ShapeDtypeStruct((nblk_max, 1), jnp.int32),
            jax.ShapeDtypeStruct((1, 1), jnp.int32),
        ],
    )(h, gate_w)

    p0v = p0.reshape(BT)
    p1v = p1.reshape(BT)
    eidv = eid.reshape(nblk_max)
    nblkv = nblk.reshape(1)

    hx0 = jnp.concatenate([h, jnp.broadcast_to(g1, (BT, _GPAD))], axis=1)
    hx1 = jnp.concatenate([h, jnp.broadcast_to(g2, (BT, _GPAD))], axis=1)

    sorted_hx = _make_scatter(NW, rows_per_w, HX, npad)(hx0, hx1, p0v, p1v)

    grid_spec = pltpu.PrefetchScalarGridSpec(
        num_scalar_prefetch=2,
        grid=(nblk_max, n_f),
        in_specs=[
            pl.BlockSpec(
                (_BM, HX),
                lambda m, f, eid, nblk: (
                    jnp.where(m < nblk[0], m, jnp.maximum(nblk[0] - 1, 0)),
                    0)),
            pl.BlockSpec(
                (1, _FB, H),
                lambda m, f, eid, nblk: (
                    eid[m], jnp.where(m < nblk[0], f, 0), 0)),
            pl.BlockSpec(
                (1, _FB, H),
                lambda m, f, eid, nblk: (
                    eid[m], jnp.where(m < nblk[0], f, 0), 0)),
            pl.BlockSpec(
                (1, H, _FB),
                lambda m, f, eid, nblk: (
                    eid[m], 0, jnp.where(m < nblk[0], f, 0))),
        ],
        out_specs=pl.BlockSpec((_BM, H), lambda m, f, eid, nblk: (m, 0)),
        scratch_shapes=[
            pltpu.VMEM((_BM, H), jnp.bfloat16),
            pltpu.VMEM((_BM, 1), jnp.float32),
        ],
    )
    y_sorted = pl.pallas_call(
        functools.partial(_gmm_kernel, h=H),
        grid_spec=grid_spec,
        out_shape=jax.ShapeDtypeStruct((npad, H), jnp.float32),
        compiler_params=pltpu.CompilerParams(
            dimension_semantics=("arbitrary", "arbitrary"),
        ),
    )(eidv, nblkv, sorted_hx, gate_proj_w, up_proj_w, down_proj_w)

    out = _make_combine(NW, rows_per_w, H)(y_sorted, p0v, p1v)
    return out.reshape(orig_shape)
